# Initial kernel scaffold; baseline (speedup 1.0000x reference)
import jax
import jax.numpy as jnp
from jax.experimental import pallas as pl


def kernel(x, edge_index, attention, Wz, bz_conv, Wr, br_conv, Wh, bh_conv, Lz, bz, Lr, br, Lh, bh, W_out, b_out):
    raise NotImplementedError("write your pallas kernel here")


# speedup vs baseline: 90.3103x; 90.3103x over previous
import functools

import jax
import jax.numpy as jnp
from jax import lax
from jax.experimental import pallas as pl
from jax.experimental.pallas import tpu as pltpu
from jax.experimental.pallas import tpu_sc as plsc

N = 50000
E = 800000
D = 64
P = 10

K = 128
W = 32
CPW = 196
EP = W * CPW * K
NP = 50176
NS = NP // 16
B = 512
G = NP // B

def _sc_deg_body(dsts, zeros1, deg_out, deg_sh, dst_idx, ones_v, tmp_v):
    c = lax.axis_index("c")
    s = lax.axis_index("s")
    wid = c * 16 + s
    for i in range(K // 16):
        ones_v[pl.ds(i * 16, 16)] = jnp.ones((16,), jnp.float32)
    pltpu.sync_copy(zeros1.at[pl.ds(s * NS, NS)], tmp_v)
    pltpu.sync_copy(tmp_v, deg_sh.at[pl.ds(s * NS, NS)])
    plsc.subcore_barrier()

    def body(t, carry):
        base = (wid * CPW + t) * K
        pltpu.sync_copy(dsts.at[pl.ds(base, K)], dst_idx)
        pltpu.sync_copy(ones_v, deg_sh.at[dst_idx], add=True)
        return carry

    lax.fori_loop(0, CPW, body, 0)
    plsc.subcore_barrier()
    pltpu.sync_copy(deg_sh.at[pl.ds(s * NS, NS)], tmp_v)
    pltpu.sync_copy(tmp_v, deg_out.at[pl.ds(c * NP + s * NS, NS)])


def _sc_agg_body(srcs, dsts, y_tab, zeros16, a_out, a_sh, src_idx, dst_idx,
                 rows_v, tmp_v, sem):
    c = lax.axis_index("c")
    s = lax.axis_index("s")
    wid = c * 16 + s
    pltpu.sync_copy(zeros16.at[pl.ds(s * NS, NS)], tmp_v)
    pltpu.sync_copy(tmp_v, a_sh.at[pl.ds(s * NS, NS)])
    plsc.subcore_barrier()

    def body(t, carry):
        base = (wid * CPW + t) * K
        pltpu.sync_copy(srcs.at[pl.ds(base, K)], src_idx)
        pltpu.sync_copy(dsts.at[pl.ds(base, K)], dst_idx)
        pltpu.async_copy(y_tab.at[src_idx], rows_v, sem).wait()
        pltpu.sync_copy(rows_v, a_sh.at[dst_idx], add=True)
        return carry

    lax.fori_loop(0, CPW, body, 0)
    plsc.subcore_barrier()
    pltpu.sync_copy(a_sh.at[pl.ds(s * NS, NS)], tmp_v)
    pltpu.sync_copy(tmp_v, a_out.at[pl.ds(c * NP + s * NS, NS)])


@functools.lru_cache(maxsize=None)
def _sc_kernels():
    mesh = plsc.VectorSubcoreMesh(core_axis_name="c", subcore_axis_name="s")
    sc_deg = pl.kernel(
        _sc_deg_body,
        mesh=mesh,
        out_type=jax.ShapeDtypeStruct((2 * NP,), jnp.float32),
        scratch_types=[
            pltpu.VMEM_SHARED((NP,), jnp.float32),
            pltpu.VMEM((K,), jnp.int32),
            pltpu.VMEM((K,), jnp.float32),
            pltpu.VMEM((NS,), jnp.float32),
        ],
    )
    sc_agg = pl.kernel(
        _sc_agg_body,
        mesh=mesh,
        out_type=jax.ShapeDtypeStruct((2 * NP, 16), jnp.float32),
        scratch_types=[
            pltpu.VMEM_SHARED((NP, 16), jnp.float32),
            pltpu.VMEM((K,), jnp.int32),
            pltpu.VMEM((K,), jnp.int32),
            pltpu.VMEM((K, 16), jnp.float32),
            pltpu.VMEM((NS, 16), jnp.float32),
            pltpu.SemaphoreType.DMA,
        ],
        compiler_params=pltpu.CompilerParams(use_tc_tiling_on_sc=False),
    )
    return sc_deg, sc_agg


def _tc_prep_body(d0, d1, xp, y, dinv):
    deg = d0[...] + d1[...] + 1.0
    dv = lax.rsqrt(deg)
    dinv[...] = dv
    y[...] = xp[...] * dv


def _tc_cell_body(a0, a1, y, dinv, Lz, Lr, Lh, Wz, Wr, Wh, bzc, brc, bhc,
                  bz, br, bh, att, Wo, bo, out):
    Lz_t, Lz_b = Lz[:D, :], Lz[D:, :]
    Lr_t, Lr_b = Lr[:D, :], Lr[D:, :]
    Lh_t, Lh_b = Lh[:D, :], Lh[D:, :]
    az = Wz[...] @ Lz_t
    ar = Wr[...] @ Lr_t
    ah = Wh[...] @ Lh_t
    kz = bzc[...] @ Lz_t + bz[...]
    kr = brc[...] @ Lr_t + br[...]
    kh = bhc[...] @ Lh_t + bh[...]
    azr = jnp.concatenate([az, ar], axis=1)
    kzr = jnp.concatenate([kz, kr], axis=1)
    Lzr = jnp.concatenate([Lz_b, Lr_b], axis=1)

    a = att[...]
    e = jnp.exp(a - jnp.max(a))
    probs = e / jnp.sum(e)

    S = dinv[...] * (a0[...] + a1[...] + y[...])
    H = jnp.zeros((B, D), jnp.float32)
    Hacc = jnp.zeros((B, D), jnp.float32)
    for p in range(P):
        sp = S[:, p:p + 1]
        zr = jax.nn.sigmoid(sp * azr + H @ Lzr + kzr)
        Z = zr[:, :D]
        R = zr[:, D:]
        ht = jnp.tanh(sp * ah + (H * R) @ Lh_b + kh)
        H = Z * H + (1.0 - Z) * ht
        Hacc = Hacc + probs[:, p:p + 1] * H
    o = jax.nn.relu(Hacc) @ Wo[...] + bo[...]
    out[...] = o[:, 0]


def kernel(x, edge_index, attention, Wz, bz_conv, Wr, br_conv, Wh, bh_conv,
           Lz, bz, Lr, br, Lh, bh, W_out, b_out):
    xr = x.reshape(N, P)
    x_pad = jnp.pad(xr, ((0, NP - N), (0, 16 - P)))
    pad = jnp.full((EP - E,), N, jnp.int32)
    srcs = jnp.concatenate([edge_index[0], pad])
    dsts = jnp.concatenate([edge_index[1], pad])
    zeros1 = jnp.zeros((NP,), jnp.float32)
    zeros16 = jnp.zeros((NP, 16), jnp.float32)

    sc_deg, sc_agg = _sc_kernels()
    deg_flat = sc_deg(dsts, zeros1)
    d0 = deg_flat[:NP].reshape(NP, 1)
    d1 = deg_flat[NP:].reshape(NP, 1)

    y, dinv = pl.pallas_call(
        _tc_prep_body,
        grid=(G,),
        in_specs=[
            pl.BlockSpec((B, 1), lambda i: (i, 0)),
            pl.BlockSpec((B, 1), lambda i: (i, 0)),
            pl.BlockSpec((B, 16), lambda i: (i, 0)),
        ],
        out_specs=[
            pl.BlockSpec((B, 16), lambda i: (i, 0)),
            pl.BlockSpec((B, 1), lambda i: (i, 0)),
        ],
        out_shape=[
            jax.ShapeDtypeStruct((NP, 16), jnp.float32),
            jax.ShapeDtypeStruct((NP, 1), jnp.float32),
        ],
    )(d0, d1, x_pad)

    a_flat = sc_agg(srcs, dsts, y, zeros16)
    a0 = a_flat[:NP]
    a1 = a_flat[NP:]

    att_pad = jnp.pad(attention, (0, 16 - P),
                      constant_values=-1e30).reshape(1, 16)
    blk = lambda r, c_: pl.BlockSpec((r, c_), lambda i: (0, 0))
    outp = pl.pallas_call(
        _tc_cell_body,
        grid=(G,),
        in_specs=[
            pl.BlockSpec((B, 16), lambda i: (i, 0)),
            pl.BlockSpec((B, 16), lambda i: (i, 0)),
            pl.BlockSpec((B, 16), lambda i: (i, 0)),
            pl.BlockSpec((B, 1), lambda i: (i, 0)),
            blk(2 * D, D), blk(2 * D, D), blk(2 * D, D),
            blk(1, D), blk(1, D), blk(1, D),
            blk(1, D), blk(1, D), blk(1, D),
            blk(1, D), blk(1, D), blk(1, D),
            blk(1, 16), blk(D, 1), blk(1, 1),
        ],
        out_specs=pl.BlockSpec((B,), lambda i: (i,)),
        out_shape=jax.ShapeDtypeStruct((NP,), jnp.float32),
    )(a0, a1, y, dinv, Lz, Lr, Lh, Wz, Wr, Wh,
      bz_conv.reshape(1, D), br_conv.reshape(1, D), bh_conv.reshape(1, D),
      bz.reshape(1, D), br.reshape(1, D), bh.reshape(1, D),
      att_pad, W_out, b_out.reshape(1, 1))

    return outp[:N]


# speedup vs baseline: 111.1287x; 1.2305x over previous
import functools

import jax
import jax.numpy as jnp
from jax import lax
from jax.experimental import pallas as pl
from jax.experimental.pallas import tpu as pltpu
from jax.experimental.pallas import tpu_sc as plsc

N = 50000
E = 800000
D = 64
P = 10

K = 128
W = 32
CPW = 196
EP = W * CPW * K
NP = 50176
NS = NP // 16
B = 512
G = NP // B

def _sc_deg_body(dsts, zeros1, deg_out, deg_sh, idx0, idx1, ones_v, tmp_v,
                 is0, is1):
    c = lax.axis_index("c")
    s = lax.axis_index("s")
    wid = c * 16 + s
    for i in range(K // 16):
        ones_v[pl.ds(i * 16, 16)] = jnp.ones((16,), jnp.float32)
    pltpu.sync_copy(zeros1.at[pl.ds(s * NS, NS)], tmp_v)
    pltpu.sync_copy(tmp_v, deg_sh.at[pl.ds(s * NS, NS)])
    plsc.subcore_barrier()

    def fire(t, idx_b, is_b):
        pltpu.async_copy(dsts.at[pl.ds(t * K, K)], idx_b, is_b)

    def wait(idx_b, is_b):
        pltpu.make_async_copy(dsts.at[pl.ds(0, K)], idx_b, is_b).wait()

    def scat(idx_b):
        pltpu.sync_copy(ones_v, deg_sh.at[idx_b], add=True)

    t0 = wid * CPW
    fire(t0, idx0, is0)
    fire(t0 + 1, idx1, is1)

    def body(j, carry):
        t = t0 + 2 * j
        wait(idx0, is0)
        scat(idx0)
        fire(t + 2, idx0, is0)
        wait(idx1, is1)
        scat(idx1)
        fire(t + 3, idx1, is1)
        return carry

    lax.fori_loop(0, CPW // 2 - 1, body, 0)
    wait(idx0, is0)
    scat(idx0)
    wait(idx1, is1)
    scat(idx1)
    plsc.subcore_barrier()
    pltpu.sync_copy(deg_sh.at[pl.ds(s * NS, NS)], tmp_v)
    pltpu.sync_copy(tmp_v, deg_out.at[pl.ds(c * NP + s * NS, NS)])


def _sc_agg_body(ei2, y_tab, zeros16, a_out, a_sh, idx0, idx1, rows0, rows1,
                 tmp_v, is0, is1, gs0, gs1):
    c = lax.axis_index("c")
    s = lax.axis_index("s")
    wid = c * 16 + s
    pltpu.sync_copy(zeros16.at[pl.ds(s * NS, NS)], tmp_v)
    pltpu.sync_copy(tmp_v, a_sh.at[pl.ds(s * NS, NS)])
    plsc.subcore_barrier()

    def fire_idx(t, idx_b, is_b):
        pltpu.async_copy(ei2.at[t], idx_b, is_b)

    def wait_idx(idx_b, is_b):
        pltpu.make_async_copy(ei2.at[0], idx_b, is_b).wait()

    def fire_gather(idx_b, rows_b, gs_b):
        pltpu.async_copy(y_tab.at[idx_b.at[0]], rows_b, gs_b)

    def wait_gather(rows_b, gs_b):
        pltpu.make_async_copy(y_tab.at[pl.ds(0, K)], rows_b, gs_b).wait()

    def scatter(idx_b, rows_b):
        pltpu.sync_copy(rows_b, a_sh.at[idx_b.at[1]], add=True)

    t0 = wid * CPW
    fire_idx(t0, idx0, is0)
    fire_idx(t0 + 1, idx1, is1)
    wait_idx(idx0, is0)
    fire_gather(idx0, rows0, gs0)
    wait_idx(idx1, is1)
    fire_gather(idx1, rows1, gs1)

    def body(j, carry):
        t = t0 + 2 * j
        wait_gather(rows0, gs0)
        scatter(idx0, rows0)
        fire_idx(t + 2, idx0, is0)
        wait_gather(rows1, gs1)
        scatter(idx1, rows1)
        fire_idx(t + 3, idx1, is1)
        wait_idx(idx0, is0)
        fire_gather(idx0, rows0, gs0)
        wait_idx(idx1, is1)
        fire_gather(idx1, rows1, gs1)
        return carry

    lax.fori_loop(0, CPW // 2 - 1, body, 0)
    wait_gather(rows0, gs0)
    scatter(idx0, rows0)
    wait_gather(rows1, gs1)
    scatter(idx1, rows1)
    plsc.subcore_barrier()
    pltpu.sync_copy(a_sh.at[pl.ds(s * NS, NS)], tmp_v)
    pltpu.sync_copy(tmp_v, a_out.at[pl.ds(c * NP + s * NS, NS)])


@functools.lru_cache(maxsize=None)
def _sc_kernels():
    mesh = plsc.VectorSubcoreMesh(core_axis_name="c", subcore_axis_name="s")
    sc_deg = pl.kernel(
        _sc_deg_body,
        mesh=mesh,
        out_type=jax.ShapeDtypeStruct((2 * NP,), jnp.float32),
        scratch_types=[
            pltpu.VMEM_SHARED((NP,), jnp.float32),
            pltpu.VMEM((K,), jnp.int32),
            pltpu.VMEM((K,), jnp.int32),
            pltpu.VMEM((K,), jnp.float32),
            pltpu.VMEM((NS,), jnp.float32),
            pltpu.SemaphoreType.DMA,
            pltpu.SemaphoreType.DMA,
        ],
    )
    sc_agg = pl.kernel(
        _sc_agg_body,
        mesh=mesh,
        out_type=jax.ShapeDtypeStruct((2 * NP, 16), jnp.float32),
        scratch_types=[
            pltpu.VMEM_SHARED((NP, 16), jnp.float32),
            pltpu.VMEM((2, K), jnp.int32),
            pltpu.VMEM((2, K), jnp.int32),
            pltpu.VMEM((K, 16), jnp.float32),
            pltpu.VMEM((K, 16), jnp.float32),
            pltpu.VMEM((NS, 16), jnp.float32),
            pltpu.SemaphoreType.DMA,
            pltpu.SemaphoreType.DMA,
            pltpu.SemaphoreType.DMA,
            pltpu.SemaphoreType.DMA,
        ],
        compiler_params=pltpu.CompilerParams(use_tc_tiling_on_sc=False),
    )
    return sc_deg, sc_agg


def _tc_prep_body(d0, d1, xp, y, d16):
    deg = d0[...] + d1[...] + 1.0
    dv = jnp.reshape(1.0 / jnp.sqrt(deg), (B, 1))
    d16[...] = jnp.broadcast_to(dv, (B, 16))
    y[...] = xp[...] * dv


def _tc_cell_body(a0, a1, y, d16, Lz, Lr, Lh, Wz, Wr, Wh, bzc, brc, bhc,
                  bz, br, bh, att, Wo, bo, out):
    Lz_t, Lz_b = Lz[:D, :], Lz[D:, :]
    Lr_t, Lr_b = Lr[:D, :], Lr[D:, :]
    Lh_t, Lh_b = Lh[:D, :], Lh[D:, :]
    Wzr, Wrr, Whr = Wz[...], Wr[...], Wh[...]
    bzcv, brcv, bhcv = bzc[...], brc[...], bhc[...]

    a = att[...]
    e = jnp.exp(a - jnp.max(a))
    probs = e / jnp.sum(e)

    S = d16[...] * (a0[...] + a1[...] + y[...])
    H = jnp.zeros((B, D), jnp.float32)
    Hacc = jnp.zeros((B, D), jnp.float32)
    for p in range(P):
        sp = S[:, p:p + 1]
        cz = sp * Wzr + bzcv
        cr = sp * Wrr + brcv
        ch = sp * Whr + bhcv
        Z = jax.nn.sigmoid(jnp.dot(cz, Lz_t) + jnp.dot(H, Lz_b) + bz[...])
        R = jax.nn.sigmoid(jnp.dot(cr, Lr_t) + jnp.dot(H, Lr_b) + br[...])
        ht = jnp.tanh(jnp.dot(ch, Lh_t) + jnp.dot(H * R, Lh_b) + bh[...])
        H = Z * H + (1.0 - Z) * ht
        Hacc = Hacc + probs[:, p:p + 1] * H
    o = jnp.dot(jax.nn.relu(Hacc), Wo[...]) + bo[...]
    out[...] = o[:, 0]


def kernel(x, edge_index, attention, Wz, bz_conv, Wr, br_conv, Wh, bh_conv,
           Lz, bz, Lr, br, Lh, bh, W_out, b_out):
    xr = x.reshape(N, P)
    x_pad = jnp.pad(xr, ((0, NP - N), (0, 16 - P)))
    pad = jnp.full((EP - E,), N, jnp.int32)
    srcs = jnp.concatenate([edge_index[0], pad])
    dsts = jnp.concatenate([edge_index[1], pad])
    CH = EP // K
    ei2 = jnp.concatenate([srcs.reshape(CH, 1, K), dsts.reshape(CH, 1, K)],
                          axis=1)
    zeros1 = jnp.zeros((NP,), jnp.float32)
    zeros16 = jnp.zeros((NP, 16), jnp.float32)

    sc_deg, sc_agg = _sc_kernels()
    deg_flat = sc_deg(dsts, zeros1)

    y, d16 = pl.pallas_call(
        _tc_prep_body,
        grid=(G,),
        in_specs=[
            pl.BlockSpec((B,), lambda i: (i,)),
            pl.BlockSpec((B,), lambda i: (i + G,)),
            pl.BlockSpec((B, 16), lambda i: (i, 0)),
        ],
        out_specs=[
            pl.BlockSpec((B, 16), lambda i: (i, 0)),
            pl.BlockSpec((B, 16), lambda i: (i, 0)),
        ],
        out_shape=[
            jax.ShapeDtypeStruct((NP, 16), jnp.float32),
            jax.ShapeDtypeStruct((NP, 16), jnp.float32),
        ],
    )(deg_flat, deg_flat, x_pad)

    a_flat = sc_agg(ei2, y, zeros16)

    att_pad = jnp.pad(attention, (0, 16 - P),
                      constant_values=-1e30).reshape(1, 16)
    blk = lambda r, c_: pl.BlockSpec((r, c_), lambda i: (0, 0))
    outp = pl.pallas_call(
        _tc_cell_body,
        grid=(G,),
        in_specs=[
            pl.BlockSpec((B, 16), lambda i: (i, 0)),
            pl.BlockSpec((B, 16), lambda i: (i + G, 0)),
            pl.BlockSpec((B, 16), lambda i: (i, 0)),
            pl.BlockSpec((B, 16), lambda i: (i, 0)),
            blk(2 * D, D), blk(2 * D, D), blk(2 * D, D),
            blk(1, D), blk(1, D), blk(1, D),
            blk(1, D), blk(1, D), blk(1, D),
            blk(1, D), blk(1, D), blk(1, D),
            blk(1, 16), blk(D, 1), blk(1, 1),
        ],
        out_specs=pl.BlockSpec((B,), lambda i: (i,)),
        out_shape=jax.ShapeDtypeStruct((N,), jnp.float32),
    )(a_flat, a_flat, y, d16, Lz, Lr, Lh, Wz, Wr, Wh,
      bz_conv.reshape(1, D), br_conv.reshape(1, D), bh_conv.reshape(1, D),
      bz.reshape(1, D), br.reshape(1, D), bh.reshape(1, D),
      att_pad, W_out, b_out.reshape(1, 1))

    return outp


# speedup vs baseline: 141.6100x; 1.2743x over previous
import functools

import jax
import jax.numpy as jnp
from jax import lax
from jax.experimental import pallas as pl
from jax.experimental.pallas import tpu as pltpu
from jax.experimental.pallas import tpu_sc as plsc

N = 50000
E = 800000
D = 64
P = 10

K = 128
W = 32
CPW = 196
EP = W * CPW * K
NP = 50176
NS = NP // 16
B = 1024
G = NP // B
BP = 7168
GP = NP // BP

def _sc_deg_body(dsts, zeros1, deg_out, deg_sh, idx0, idx1, ones_v, tmp_v,
                 is0, is1):
    c = lax.axis_index("c")
    s = lax.axis_index("s")
    wid = c * 16 + s
    for i in range(K // 16):
        ones_v[pl.ds(i * 16, 16)] = jnp.ones((16,), jnp.float32)
    pltpu.sync_copy(zeros1.at[pl.ds(s * NS, NS)], tmp_v)
    pltpu.sync_copy(tmp_v, deg_sh.at[pl.ds(s * NS, NS)])
    plsc.subcore_barrier()

    def fire(t, idx_b, is_b):
        pltpu.async_copy(dsts.at[pl.ds(t * K, K)], idx_b, is_b)

    def wait(idx_b, is_b):
        pltpu.make_async_copy(dsts.at[pl.ds(0, K)], idx_b, is_b).wait()

    def scat(idx_b):
        pltpu.sync_copy(ones_v, deg_sh.at[idx_b], add=True)

    t0 = wid * CPW
    fire(t0, idx0, is0)
    fire(t0 + 1, idx1, is1)

    def body(j, carry):
        t = t0 + 2 * j
        wait(idx0, is0)
        scat(idx0)
        fire(t + 2, idx0, is0)
        wait(idx1, is1)
        scat(idx1)
        fire(t + 3, idx1, is1)
        return carry

    lax.fori_loop(0, CPW // 2 - 1, body, 0)
    wait(idx0, is0)
    scat(idx0)
    wait(idx1, is1)
    scat(idx1)
    plsc.subcore_barrier()
    pltpu.sync_copy(deg_sh.at[pl.ds(s * NS, NS)], tmp_v)
    pltpu.sync_copy(tmp_v, deg_out.at[pl.ds(c * NP + s * NS, NS)])


def _sc_agg_body(ei2, y_tab, zeros16, a_out, a_sh, idx0, idx1, rows0, rows1,
                 tmp_v, is0, is1, gs0, gs1):
    c = lax.axis_index("c")
    s = lax.axis_index("s")
    wid = c * 16 + s
    pltpu.sync_copy(zeros16.at[pl.ds(s * NS, NS)], tmp_v)
    pltpu.sync_copy(tmp_v, a_sh.at[pl.ds(s * NS, NS)])
    plsc.subcore_barrier()

    def fire_idx(t, idx_b, is_b):
        pltpu.async_copy(ei2.at[t], idx_b, is_b)

    def wait_idx(idx_b, is_b):
        pltpu.make_async_copy(ei2.at[0], idx_b, is_b).wait()

    def fire_gather(idx_b, rows_b, gs_b):
        pltpu.async_copy(y_tab.at[idx_b.at[0]], rows_b, gs_b)

    def wait_gather(rows_b, gs_b):
        pltpu.make_async_copy(y_tab.at[pl.ds(0, K)], rows_b, gs_b).wait()

    def scatter(idx_b, rows_b):
        pltpu.sync_copy(rows_b, a_sh.at[idx_b.at[1]], add=True)

    t0 = wid * CPW
    fire_idx(t0, idx0, is0)
    fire_idx(t0 + 1, idx1, is1)
    wait_idx(idx0, is0)
    fire_gather(idx0, rows0, gs0)
    wait_idx(idx1, is1)
    fire_gather(idx1, rows1, gs1)

    def body(j, carry):
        t = t0 + 2 * j
        wait_gather(rows0, gs0)
        scatter(idx0, rows0)
        fire_idx(t + 2, idx0, is0)
        wait_gather(rows1, gs1)
        scatter(idx1, rows1)
        fire_idx(t + 3, idx1, is1)
        wait_idx(idx0, is0)
        fire_gather(idx0, rows0, gs0)
        wait_idx(idx1, is1)
        fire_gather(idx1, rows1, gs1)
        return carry

    lax.fori_loop(0, CPW // 2 - 1, body, 0)
    wait_gather(rows0, gs0)
    scatter(idx0, rows0)
    wait_gather(rows1, gs1)
    scatter(idx1, rows1)
    plsc.subcore_barrier()
    pltpu.sync_copy(a_sh.at[pl.ds(s * NS, NS)], tmp_v)
    pltpu.sync_copy(tmp_v, a_out.at[pl.ds(c * NP + s * NS, NS)])


@functools.lru_cache(maxsize=None)
def _sc_kernels():
    mesh = plsc.VectorSubcoreMesh(core_axis_name="c", subcore_axis_name="s")
    sc_deg = pl.kernel(
        _sc_deg_body,
        mesh=mesh,
        out_type=jax.ShapeDtypeStruct((2 * NP,), jnp.float32),
        scratch_types=[
            pltpu.VMEM_SHARED((NP,), jnp.float32),
            pltpu.VMEM((K,), jnp.int32),
            pltpu.VMEM((K,), jnp.int32),
            pltpu.VMEM((K,), jnp.float32),
            pltpu.VMEM((NS,), jnp.float32),
            pltpu.SemaphoreType.DMA,
            pltpu.SemaphoreType.DMA,
        ],
    )
    sc_agg = pl.kernel(
        _sc_agg_body,
        mesh=mesh,
        out_type=jax.ShapeDtypeStruct((2 * NP, 16), jnp.float32),
        scratch_types=[
            pltpu.VMEM_SHARED((NP, 16), jnp.float32),
            pltpu.VMEM((2, K), jnp.int32),
            pltpu.VMEM((2, K), jnp.int32),
            pltpu.VMEM((K, 16), jnp.float32),
            pltpu.VMEM((K, 16), jnp.float32),
            pltpu.VMEM((NS, 16), jnp.float32),
            pltpu.SemaphoreType.DMA,
            pltpu.SemaphoreType.DMA,
            pltpu.SemaphoreType.DMA,
            pltpu.SemaphoreType.DMA,
        ],
        compiler_params=pltpu.CompilerParams(use_tc_tiling_on_sc=False),
    )
    return sc_deg, sc_agg


def _tc_prep_body(d0, d1, xp, y, d16):
    deg = d0[...] + d1[...] + 1.0
    dv = jnp.reshape(1.0 / jnp.sqrt(deg), (BP, 1))
    d16[...] = jnp.broadcast_to(dv, (BP, 16))
    y[...] = xp[...] * dv


def _tc_cell_body(a0, a1, y, d16, Lz, Lr, Lh, Wz, Wr, Wh, bzc, brc, bhc,
                  bz, br, bh, att, Wo, bo, out):
    Lz_t, Lz_b = Lz[:D, :], Lz[D:, :]
    Lr_t, Lr_b = Lr[:D, :], Lr[D:, :]
    Lh_t, Lh_b = Lh[:D, :], Lh[D:, :]
    Wzr, Wrr, Whr = Wz[...], Wr[...], Wh[...]
    bzcv, brcv, bhcv = bzc[...], brc[...], bhc[...]

    a = att[...]
    e = jnp.exp(a - jnp.max(a))
    probs = e / jnp.sum(e)

    S = d16[...] * (a0[...] + a1[...] + y[...])
    H = jnp.zeros((B, D), jnp.float32)
    Hacc = jnp.zeros((B, D), jnp.float32)
    for p in range(P):
        sp = S[:, p:p + 1]
        cz = sp * Wzr + bzcv
        cr = sp * Wrr + brcv
        ch = sp * Whr + bhcv
        Z = jax.nn.sigmoid(jnp.dot(cz, Lz_t) + jnp.dot(H, Lz_b) + bz[...])
        R = jax.nn.sigmoid(jnp.dot(cr, Lr_t) + jnp.dot(H, Lr_b) + br[...])
        ht = jnp.tanh(jnp.dot(ch, Lh_t) + jnp.dot(H * R, Lh_b) + bh[...])
        H = ht + Z * (H - ht)
        Hacc = Hacc + probs[:, p:p + 1] * H
    o = jnp.dot(jax.nn.relu(Hacc), Wo[...]) + bo[...]
    out[...] = o[:, 0]


def kernel(x, edge_index, attention, Wz, bz_conv, Wr, br_conv, Wh, bh_conv,
           Lz, bz, Lr, br, Lh, bh, W_out, b_out):
    xr = x.reshape(N, P)
    x_pad = jnp.pad(xr, ((0, NP - N), (0, 16 - P)))
    pad = jnp.full((EP - E,), N, jnp.int32)
    srcs = jnp.concatenate([edge_index[0], pad])
    dsts = jnp.concatenate([edge_index[1], pad])
    CH = EP // K
    ei2 = jnp.concatenate([srcs.reshape(CH, 1, K), dsts.reshape(CH, 1, K)],
                          axis=1)
    zeros1 = jnp.zeros((NP,), jnp.float32)
    zeros16 = jnp.zeros((NP, 16), jnp.float32)

    sc_deg, sc_agg = _sc_kernels()
    deg_flat = sc_deg(dsts, zeros1)

    y, d16 = pl.pallas_call(
        _tc_prep_body,
        grid=(GP,),
        in_specs=[
            pl.BlockSpec((BP,), lambda i: (i,)),
            pl.BlockSpec((BP,), lambda i: (i + GP,)),
            pl.BlockSpec((BP, 16), lambda i: (i, 0)),
        ],
        out_specs=[
            pl.BlockSpec((BP, 16), lambda i: (i, 0)),
            pl.BlockSpec((BP, 16), lambda i: (i, 0)),
        ],
        out_shape=[
            jax.ShapeDtypeStruct((NP, 16), jnp.float32),
            jax.ShapeDtypeStruct((NP, 16), jnp.float32),
        ],
    )(deg_flat, deg_flat, x_pad)

    a_flat = sc_agg(ei2, y, zeros16)

    att_pad = jnp.pad(attention, (0, 16 - P),
                      constant_values=-1e30).reshape(1, 16)
    blk = lambda r, c_: pl.BlockSpec((r, c_), lambda i: (0, 0))
    outp = pl.pallas_call(
        _tc_cell_body,
        grid=(G,),
        in_specs=[
            pl.BlockSpec((B, 16), lambda i: (i, 0)),
            pl.BlockSpec((B, 16), lambda i: (i + G, 0)),
            pl.BlockSpec((B, 16), lambda i: (i, 0)),
            pl.BlockSpec((B, 16), lambda i: (i, 0)),
            blk(2 * D, D), blk(2 * D, D), blk(2 * D, D),
            blk(1, D), blk(1, D), blk(1, D),
            blk(1, D), blk(1, D), blk(1, D),
            blk(1, D), blk(1, D), blk(1, D),
            blk(1, 16), blk(D, 1), blk(1, 1),
        ],
        out_specs=pl.BlockSpec((B,), lambda i: (i,)),
        out_shape=jax.ShapeDtypeStruct((N,), jnp.float32),
    )(a_flat, a_flat, y, d16, Lz, Lr, Lh, Wz, Wr, Wh,
      bz_conv.reshape(1, D), br_conv.reshape(1, D), bh_conv.reshape(1, D),
      bz.reshape(1, D), br.reshape(1, D), bh.reshape(1, D),
      att_pad, W_out, b_out.reshape(1, 1))

    return outp


# speedup vs baseline: 147.2034x; 1.0395x over previous
import functools

import jax
import jax.numpy as jnp
from jax import lax
from jax.experimental import pallas as pl
from jax.experimental.pallas import tpu as pltpu
from jax.experimental.pallas import tpu_sc as plsc

N = 50000
E = 800000
D = 64
P = 10

K = 128
W = 32
NCH = E // K
CPW = 196
NP = 50176
NS = NP // 16
B = 1024
G = NP // B
BP = 7168
GP = NP // BP

def _sc_deg_body(ei2d, zeros1, deg_out, deg_sh, idx0, idx1, ones_v, tmp_v,
                 is0, is1):
    c = lax.axis_index("c")
    s = lax.axis_index("s")
    wid = c * 16 + s
    for i in range(K // 16):
        ones_v[pl.ds(i * 16, 16)] = jnp.ones((16,), jnp.float32)
    pltpu.sync_copy(zeros1.at[pl.ds(s * NS, NS)], tmp_v)
    pltpu.sync_copy(tmp_v, deg_sh.at[pl.ds(s * NS, NS)])
    plsc.subcore_barrier()

    def fire(t, idx_b, is_b):
        @pl.when(t < NCH)
        def _():
            pltpu.async_copy(ei2d.at[NCH + t], idx_b, is_b)

    def drain(t, idx_b, is_b):
        @pl.when(t < NCH)
        def _():
            pltpu.make_async_copy(ei2d.at[0], idx_b, is_b).wait()
            pltpu.sync_copy(ones_v, deg_sh.at[idx_b], add=True)

    t0 = wid * CPW
    fire(t0, idx0, is0)
    fire(t0 + 1, idx1, is1)

    def body(j, carry):
        t = t0 + 2 * j
        drain(t, idx0, is0)
        fire(t + 2, idx0, is0)
        drain(t + 1, idx1, is1)
        fire(t + 3, idx1, is1)
        return carry

    lax.fori_loop(0, CPW // 2 - 1, body, 0)
    drain(t0 + CPW - 2, idx0, is0)
    drain(t0 + CPW - 1, idx1, is1)
    plsc.subcore_barrier()
    pltpu.sync_copy(deg_sh.at[pl.ds(s * NS, NS)], tmp_v)
    pltpu.sync_copy(tmp_v, deg_out.at[pl.ds(c * NP + s * NS, NS)])


def _sc_agg_body(ei2d, y_tab, zeros16, a_out, a_sh, ids0, idd0, ids1, idd1,
                 rows0, rows1, tmp_v, is0, is1, gs0, gs1):
    c = lax.axis_index("c")
    s = lax.axis_index("s")
    wid = c * 16 + s
    pltpu.sync_copy(zeros16.at[pl.ds(s * NS, NS)], tmp_v)
    pltpu.sync_copy(tmp_v, a_sh.at[pl.ds(s * NS, NS)])
    plsc.subcore_barrier()

    def fire_idx(t, ids_b, idd_b, is_b):
        @pl.when(t < NCH)
        def _():
            pltpu.async_copy(ei2d.at[t], ids_b, is_b)
            pltpu.async_copy(ei2d.at[NCH + t], idd_b, is_b)

    def start_gather(t, ids_b, idd_b, rows_b, is_b, gs_b):
        @pl.when(t < NCH)
        def _():
            pltpu.make_async_copy(ei2d.at[0], ids_b, is_b).wait()
            pltpu.make_async_copy(ei2d.at[0], idd_b, is_b).wait()
            pltpu.async_copy(y_tab.at[ids_b], rows_b, gs_b)

    def drain(t, idd_b, rows_b, gs_b):
        @pl.when(t < NCH)
        def _():
            pltpu.make_async_copy(y_tab.at[pl.ds(0, K)], rows_b, gs_b).wait()
            pltpu.sync_copy(rows_b, a_sh.at[idd_b], add=True)

    t0 = wid * CPW
    fire_idx(t0, ids0, idd0, is0)
    fire_idx(t0 + 1, ids1, idd1, is1)
    start_gather(t0, ids0, idd0, rows0, is0, gs0)
    start_gather(t0 + 1, ids1, idd1, rows1, is1, gs1)

    def body(j, carry):
        t = t0 + 2 * j
        drain(t, idd0, rows0, gs0)
        fire_idx(t + 2, ids0, idd0, is0)
        drain(t + 1, idd1, rows1, gs1)
        fire_idx(t + 3, ids1, idd1, is1)
        start_gather(t + 2, ids0, idd0, rows0, is0, gs0)
        start_gather(t + 3, ids1, idd1, rows1, is1, gs1)
        return carry

    lax.fori_loop(0, CPW // 2 - 1, body, 0)
    drain(t0 + CPW - 2, idd0, rows0, gs0)
    drain(t0 + CPW - 1, idd1, rows1, gs1)
    plsc.subcore_barrier()
    pltpu.sync_copy(a_sh.at[pl.ds(s * NS, NS)], tmp_v)
    pltpu.sync_copy(tmp_v, a_out.at[pl.ds(c * NP + s * NS, NS)])


@functools.lru_cache(maxsize=None)
def _sc_kernels():
    mesh = plsc.VectorSubcoreMesh(core_axis_name="c", subcore_axis_name="s")
    sc_deg = pl.kernel(
        _sc_deg_body,
        mesh=mesh,
        out_type=jax.ShapeDtypeStruct((2 * NP,), jnp.float32),
        scratch_types=[
            pltpu.VMEM_SHARED((NP,), jnp.float32),
            pltpu.VMEM((K,), jnp.int32),
            pltpu.VMEM((K,), jnp.int32),
            pltpu.VMEM((K,), jnp.float32),
            pltpu.VMEM((NS,), jnp.float32),
            pltpu.SemaphoreType.DMA,
            pltpu.SemaphoreType.DMA,
        ],
    )
    sc_agg = pl.kernel(
        _sc_agg_body,
        mesh=mesh,
        out_type=jax.ShapeDtypeStruct((2 * NP, 16), jnp.float32),
        scratch_types=[
            pltpu.VMEM_SHARED((NP, 16), jnp.float32),
            pltpu.VMEM((K,), jnp.int32),
            pltpu.VMEM((K,), jnp.int32),
            pltpu.VMEM((K,), jnp.int32),
            pltpu.VMEM((K,), jnp.int32),
            pltpu.VMEM((K, 16), jnp.float32),
            pltpu.VMEM((K, 16), jnp.float32),
            pltpu.VMEM((NS, 16), jnp.float32),
            pltpu.SemaphoreType.DMA,
            pltpu.SemaphoreType.DMA,
            pltpu.SemaphoreType.DMA,
            pltpu.SemaphoreType.DMA,
        ],
        compiler_params=pltpu.CompilerParams(use_tc_tiling_on_sc=False),
    )
    return sc_deg, sc_agg


def _tc_prep_body(d0, d1, xp, y, d16):
    deg = d0[...] + d1[...] + 1.0
    dv = jnp.reshape(1.0 / jnp.sqrt(deg), (BP, 1))
    d16[...] = jnp.broadcast_to(dv, (BP, 16))
    y[...] = xp[...] * dv


def _tc_cell_body(a0, a1, y, d16, Lz, Lr, Lh, Wz, Wr, Wh, bzc, brc, bhc,
                  bz, br, bh, att, Wo, bo, out):
    Lz_t, Lz_b = Lz[:D, :], Lz[D:, :]
    Lr_t, Lr_b = Lr[:D, :], Lr[D:, :]
    Lh_t, Lh_b = Lh[:D, :], Lh[D:, :]
    Wzr, Wrr, Whr = Wz[...], Wr[...], Wh[...]
    bzcv, brcv, bhcv = bzc[...], brc[...], bhc[...]

    a = att[...]
    e = jnp.exp(a - jnp.max(a))
    probs = e / jnp.sum(e)

    S = d16[...] * (a0[...] + a1[...] + y[...])
    H = jnp.zeros((B, D), jnp.float32)
    Hacc = jnp.zeros((B, D), jnp.float32)
    for p in range(P):
        sp = S[:, p:p + 1]
        cz = sp * Wzr + bzcv
        cr = sp * Wrr + brcv
        ch = sp * Whr + bhcv
        Z = jax.nn.sigmoid(jnp.dot(cz, Lz_t) + jnp.dot(H, Lz_b) + bz[...])
        R = jax.nn.sigmoid(jnp.dot(cr, Lr_t) + jnp.dot(H, Lr_b) + br[...])
        ht = jnp.tanh(jnp.dot(ch, Lh_t) + jnp.dot(H * R, Lh_b) + bh[...])
        H = ht + Z * (H - ht)
        Hacc = Hacc + probs[:, p:p + 1] * H
    o = jnp.dot(jax.nn.relu(Hacc), Wo[...]) + bo[...]
    out[...] = o[:, 0]


def kernel(x, edge_index, attention, Wz, bz_conv, Wr, br_conv, Wh, bh_conv,
           Lz, bz, Lr, br, Lh, bh, W_out, b_out):
    xr = x.reshape(N, P)
    x_pad = jnp.pad(xr, ((0, NP - N), (0, 16 - P)))
    ei2d = edge_index.reshape(2 * NCH, K)
    zeros1 = jnp.zeros((NP,), jnp.float32)
    zeros16 = jnp.zeros((NP, 16), jnp.float32)

    sc_deg, sc_agg = _sc_kernels()
    deg_flat = sc_deg(ei2d, zeros1)

    y, d16 = pl.pallas_call(
        _tc_prep_body,
        grid=(GP,),
        in_specs=[
            pl.BlockSpec((BP,), lambda i: (i,)),
            pl.BlockSpec((BP,), lambda i: (i + GP,)),
            pl.BlockSpec((BP, 16), lambda i: (i, 0)),
        ],
        out_specs=[
            pl.BlockSpec((BP, 16), lambda i: (i, 0)),
            pl.BlockSpec((BP, 16), lambda i: (i, 0)),
        ],
        out_shape=[
            jax.ShapeDtypeStruct((NP, 16), jnp.float32),
            jax.ShapeDtypeStruct((NP, 16), jnp.float32),
        ],
    )(deg_flat, deg_flat, x_pad)

    a_flat = sc_agg(ei2d, y, zeros16)

    att_pad = jnp.pad(attention, (0, 16 - P),
                      constant_values=-1e30).reshape(1, 16)
    blk = lambda r, c_: pl.BlockSpec((r, c_), lambda i: (0, 0))
    outp = pl.pallas_call(
        _tc_cell_body,
        grid=(G,),
        in_specs=[
            pl.BlockSpec((B, 16), lambda i: (i, 0)),
            pl.BlockSpec((B, 16), lambda i: (i + G, 0)),
            pl.BlockSpec((B, 16), lambda i: (i, 0)),
            pl.BlockSpec((B, 16), lambda i: (i, 0)),
            blk(2 * D, D), blk(2 * D, D), blk(2 * D, D),
            blk(1, D), blk(1, D), blk(1, D),
            blk(1, D), blk(1, D), blk(1, D),
            blk(1, D), blk(1, D), blk(1, D),
            blk(1, 16), blk(D, 1), blk(1, 1),
        ],
        out_specs=pl.BlockSpec((B,), lambda i: (i,)),
        out_shape=jax.ShapeDtypeStruct((N,), jnp.float32),
    )(a_flat, a_flat, y, d16, Lz, Lr, Lh, Wz, Wr, Wh,
      bz_conv.reshape(1, D), br_conv.reshape(1, D), bh_conv.reshape(1, D),
      bz.reshape(1, D), br.reshape(1, D), bh.reshape(1, D),
      att_pad, W_out, b_out.reshape(1, 1))

    return outp


# speedup vs baseline: 156.8262x; 1.0654x over previous
import functools

import jax
import jax.numpy as jnp
from jax import lax
from jax.experimental import pallas as pl
from jax.experimental.pallas import tpu as pltpu
from jax.experimental.pallas import tpu_sc as plsc

N = 50000
E = 800000
D = 64
P = 10

K = 128
W = 32
NCH = E // K
CPW = 196
NP = 51200
NS = NP // 16
B = 2048
G = NP // B
BP = 2048
GP = NP // BP

def _sc_deg_body(ei2d, zeros1, deg_out, deg_sh, idx0, idx1, ones_v, tmp_v,
                 is0, is1):
    c = lax.axis_index("c")
    s = lax.axis_index("s")
    wid = c * 16 + s
    for i in range(K // 16):
        ones_v[pl.ds(i * 16, 16)] = jnp.ones((16,), jnp.float32)
    pltpu.sync_copy(zeros1.at[pl.ds(s * NS, NS)], tmp_v)
    pltpu.sync_copy(tmp_v, deg_sh.at[pl.ds(s * NS, NS)])
    plsc.subcore_barrier()

    def fire(t, idx_b, is_b):
        @pl.when(t < NCH)
        def _():
            pltpu.async_copy(ei2d.at[NCH + t], idx_b, is_b)

    def drain(t, idx_b, is_b):
        @pl.when(t < NCH)
        def _():
            pltpu.make_async_copy(ei2d.at[0], idx_b, is_b).wait()
            pltpu.sync_copy(ones_v, deg_sh.at[idx_b], add=True)

    t0 = wid * CPW
    fire(t0, idx0, is0)
    fire(t0 + 1, idx1, is1)

    def body(j, carry):
        t = t0 + 2 * j
        drain(t, idx0, is0)
        fire(t + 2, idx0, is0)
        drain(t + 1, idx1, is1)
        fire(t + 3, idx1, is1)
        return carry

    lax.fori_loop(0, CPW // 2 - 1, body, 0)
    drain(t0 + CPW - 2, idx0, is0)
    drain(t0 + CPW - 1, idx1, is1)
    plsc.subcore_barrier()
    pltpu.sync_copy(deg_sh.at[pl.ds(s * NS, NS)], tmp_v)
    pltpu.sync_copy(tmp_v, deg_out.at[pl.ds(c * NP + s * NS, NS)])


def _sc_agg_body(ei2d, y_tab, zeros16, a_out, a_sh, ids0, idd0, ids1, idd1,
                 rows0, rows1, tmp_v, is0, is1, gs0, gs1):
    c = lax.axis_index("c")
    s = lax.axis_index("s")
    wid = c * 16 + s
    pltpu.sync_copy(zeros16.at[pl.ds(s * NS, NS)], tmp_v)
    pltpu.sync_copy(tmp_v, a_sh.at[pl.ds(s * NS, NS)])
    plsc.subcore_barrier()

    def fire_idx(t, ids_b, idd_b, is_b):
        @pl.when(t < NCH)
        def _():
            pltpu.async_copy(ei2d.at[t], ids_b, is_b)
            pltpu.async_copy(ei2d.at[NCH + t], idd_b, is_b)

    def start_gather(t, ids_b, idd_b, rows_b, is_b, gs_b):
        @pl.when(t < NCH)
        def _():
            pltpu.make_async_copy(ei2d.at[0], ids_b, is_b).wait()
            pltpu.make_async_copy(ei2d.at[0], idd_b, is_b).wait()
            pltpu.async_copy(y_tab.at[ids_b], rows_b, gs_b)

    def drain(t, idd_b, rows_b, gs_b):
        @pl.when(t < NCH)
        def _():
            pltpu.make_async_copy(y_tab.at[pl.ds(0, K)], rows_b, gs_b).wait()
            pltpu.sync_copy(rows_b, a_sh.at[idd_b], add=True)

    t0 = wid * CPW
    fire_idx(t0, ids0, idd0, is0)
    fire_idx(t0 + 1, ids1, idd1, is1)
    start_gather(t0, ids0, idd0, rows0, is0, gs0)
    start_gather(t0 + 1, ids1, idd1, rows1, is1, gs1)

    def body(j, carry):
        t = t0 + 2 * j
        drain(t, idd0, rows0, gs0)
        fire_idx(t + 2, ids0, idd0, is0)
        drain(t + 1, idd1, rows1, gs1)
        fire_idx(t + 3, ids1, idd1, is1)
        start_gather(t + 2, ids0, idd0, rows0, is0, gs0)
        start_gather(t + 3, ids1, idd1, rows1, is1, gs1)
        return carry

    lax.fori_loop(0, CPW // 2 - 1, body, 0)
    drain(t0 + CPW - 2, idd0, rows0, gs0)
    drain(t0 + CPW - 1, idd1, rows1, gs1)
    plsc.subcore_barrier()
    pltpu.sync_copy(a_sh.at[pl.ds(s * NS, NS)], tmp_v)
    pltpu.sync_copy(tmp_v, a_out.at[pl.ds(c * NP + s * NS, NS)])


@functools.lru_cache(maxsize=None)
def _sc_kernels():
    mesh = plsc.VectorSubcoreMesh(core_axis_name="c", subcore_axis_name="s")
    sc_deg = pl.kernel(
        _sc_deg_body,
        mesh=mesh,
        out_type=jax.ShapeDtypeStruct((2 * NP,), jnp.float32),
        scratch_types=[
            pltpu.VMEM_SHARED((NP,), jnp.float32),
            pltpu.VMEM((K,), jnp.int32),
            pltpu.VMEM((K,), jnp.int32),
            pltpu.VMEM((K,), jnp.float32),
            pltpu.VMEM((NS,), jnp.float32),
            pltpu.SemaphoreType.DMA,
            pltpu.SemaphoreType.DMA,
        ],
    )
    sc_agg = pl.kernel(
        _sc_agg_body,
        mesh=mesh,
        out_type=jax.ShapeDtypeStruct((2 * NP, 16), jnp.float32),
        scratch_types=[
            pltpu.VMEM_SHARED((NP, 16), jnp.float32),
            pltpu.VMEM((K,), jnp.int32),
            pltpu.VMEM((K,), jnp.int32),
            pltpu.VMEM((K,), jnp.int32),
            pltpu.VMEM((K,), jnp.int32),
            pltpu.VMEM((K, 16), jnp.float32),
            pltpu.VMEM((K, 16), jnp.float32),
            pltpu.VMEM((NS, 16), jnp.float32),
            pltpu.SemaphoreType.DMA,
            pltpu.SemaphoreType.DMA,
            pltpu.SemaphoreType.DMA,
            pltpu.SemaphoreType.DMA,
        ],
        compiler_params=pltpu.CompilerParams(use_tc_tiling_on_sc=False),
    )
    return sc_deg, sc_agg


def _tc_prep_body(d0, d1, xp, y, d16):
    deg = d0[...] + d1[...] + 1.0
    dv = jnp.reshape(1.0 / jnp.sqrt(deg), (BP, 1))
    d16[...] = jnp.broadcast_to(dv, (BP, 16))
    y[...] = xp[...] * dv


def _tc_cell_body(a0, a1, y, d16, Lz, Lr, Lh, Wz, Wr, Wh, bzc, brc, bhc,
                  bz, br, bh, att, Wo, bo, out):
    Lz_t, Lz_b = Lz[:D, :], Lz[D:, :]
    Lr_t, Lr_b = Lr[:D, :], Lr[D:, :]
    Lh_t, Lh_b = Lh[:D, :], Lh[D:, :]
    Wzr, Wrr, Whr = Wz[...], Wr[...], Wh[...]
    bzcv, brcv, bhcv = bzc[...], brc[...], bhc[...]

    a = att[...]
    e = jnp.exp(a - jnp.max(a))
    probs = e / jnp.sum(e)

    S = d16[...] * (a0[...] + a1[...] + y[...])
    H = jnp.zeros((B, D), jnp.float32)
    Hacc = jnp.zeros((B, D), jnp.float32)
    for p in range(P):
        sp = S[:, p:p + 1]
        cz = sp * Wzr + bzcv
        cr = sp * Wrr + brcv
        ch = sp * Whr + bhcv
        Z = jax.nn.sigmoid(jnp.dot(cz, Lz_t) + jnp.dot(H, Lz_b) + bz[...])
        R = jax.nn.sigmoid(jnp.dot(cr, Lr_t) + jnp.dot(H, Lr_b) + br[...])
        ht = jnp.tanh(jnp.dot(ch, Lh_t) + jnp.dot(H * R, Lh_b) + bh[...])
        H = ht + Z * (H - ht)
        Hacc = Hacc + probs[:, p:p + 1] * H
    o = jnp.dot(jax.nn.relu(Hacc), Wo[...]) + bo[...]
    out[...] = o[:, 0]


def kernel(x, edge_index, attention, Wz, bz_conv, Wr, br_conv, Wh, bh_conv,
           Lz, bz, Lr, br, Lh, bh, W_out, b_out):
    xr = x.reshape(N, P)
    x_pad = jnp.pad(xr, ((0, NP - N), (0, 16 - P)))
    ei2d = edge_index.reshape(2 * NCH, K)
    zeros1 = jnp.zeros((NP,), jnp.float32)
    zeros16 = jnp.zeros((NP, 16), jnp.float32)

    sc_deg, sc_agg = _sc_kernels()
    deg_flat = sc_deg(ei2d, zeros1)

    y, d16 = pl.pallas_call(
        _tc_prep_body,
        grid=(GP,),
        in_specs=[
            pl.BlockSpec((BP,), lambda i: (i,)),
            pl.BlockSpec((BP,), lambda i: (i + GP,)),
            pl.BlockSpec((BP, 16), lambda i: (i, 0)),
        ],
        out_specs=[
            pl.BlockSpec((BP, 16), lambda i: (i, 0)),
            pl.BlockSpec((BP, 16), lambda i: (i, 0)),
        ],
        out_shape=[
            jax.ShapeDtypeStruct((NP, 16), jnp.float32),
            jax.ShapeDtypeStruct((NP, 16), jnp.float32),
        ],
    )(deg_flat, deg_flat, x_pad)

    a_flat = sc_agg(ei2d, y, zeros16)

    att_pad = jnp.pad(attention, (0, 16 - P),
                      constant_values=-1e30).reshape(1, 16)
    blk = lambda r, c_: pl.BlockSpec((r, c_), lambda i: (0, 0))
    outp = pl.pallas_call(
        _tc_cell_body,
        grid=(G,),
        in_specs=[
            pl.BlockSpec((B, 16), lambda i: (i, 0)),
            pl.BlockSpec((B, 16), lambda i: (i + G, 0)),
            pl.BlockSpec((B, 16), lambda i: (i, 0)),
            pl.BlockSpec((B, 16), lambda i: (i, 0)),
            blk(2 * D, D), blk(2 * D, D), blk(2 * D, D),
            blk(1, D), blk(1, D), blk(1, D),
            blk(1, D), blk(1, D), blk(1, D),
            blk(1, D), blk(1, D), blk(1, D),
            blk(1, 16), blk(D, 1), blk(1, 1),
        ],
        out_specs=pl.BlockSpec((B,), lambda i: (i,)),
        out_shape=jax.ShapeDtypeStruct((N,), jnp.float32),
    )(a_flat, a_flat, y, d16, Lz, Lr, Lh, Wz, Wr, Wh,
      bz_conv.reshape(1, D), br_conv.reshape(1, D), bh_conv.reshape(1, D),
      bz.reshape(1, D), br.reshape(1, D), bh.reshape(1, D),
      att_pad, W_out, b_out.reshape(1, 1))

    return outp
